# baseline (speedup 1.0000x reference)
import jax
import jax.numpy as jnp
from jax import lax
from jax.experimental import pallas as pl
from jax.experimental.pallas import tpu as pltpu

_CONF_THRES = 0.001
_IOU_THRES = 0.45
_NC = 4
_MAX_WH = 4096.0
_MAX_DET = 300

_N = 5000
_NPAD = 5120
_ROWS = 40
_LANES = 128

_NF = 10


def _nms_body(x_ref, o_ref, f_ref):
    nb = x_ref.shape[0]

    def fld(c):
        return x_ref[:, c, :].reshape(-1, _ROWS, _LANES)

    cx, cy, w, h = fld(0), fld(1), fld(2), fld(3)
    obj = fld(4)
    x1 = cx - w / 2.0
    y1 = cy - h / 2.0
    x2 = cx + w / 2.0
    y2 = cy + h / 2.0

    c0 = fld(5) * obj
    c1 = fld(6) * obj
    c2 = fld(7) * obj
    c3 = fld(8) * obj
    conf = jnp.maximum(jnp.maximum(c0, c1), jnp.maximum(c2, c3))
    jf = jnp.where(
        c0 == conf,
        0.0,
        jnp.where(c1 == conf, 1.0, jnp.where(c2 == conf, 2.0, 3.0)),
    )

    idx = lax.broadcasted_iota(jnp.int32, (_ROWS, _LANES), 0) * _LANES + lax.broadcasted_iota(
        jnp.int32, (_ROWS, _LANES), 1
    )
    pad = idx >= _N
    scores0 = jnp.where((conf > _CONF_THRES) & (~pad[None]), conf, -1.0)

    off = jf * _MAX_WH
    x1o = x1 + off
    y1o = y1 + off
    x2o = x2 + off
    y2o = y2 + off
    areao = (x2o - x1o) * (y2o - y1o)

    for f, arr in enumerate([x1o, y1o, x2o, y2o, areao, x1, y1, x2, y2, jf]):
        f_ref[f] = arr.reshape(nb * _ROWS, _LANES)

    neg = jnp.float32(-jnp.inf)
    big = jnp.int32(1 << 30)
    lane = lax.broadcasted_iota(jnp.int32, (1, _LANES), 1)

    xo = [x1o, y1o, x2o, y2o, areao]

    def it(t, carry):
        new = []
        for b in range(nb):
            sc = carry[b]
            s = jnp.max(sc)
            m = jnp.min(jnp.where(sc == s, idx, big))
            r = m // _LANES
            c = m % _LANES
            vals = f_ref[:, pl.ds(b * _ROWS + r, 1), :]
            ext = jnp.sum(jnp.where((lane == c)[None], vals, 0.0), axis=2)

            bx1o = ext[0, 0]
            by1o = ext[1, 0]
            bx2o = ext[2, 0]
            by2o = ext[3, 0]
            barea = ext[4, 0]

            x1ob, y1ob, x2ob, y2ob, areaob = (a[b] for a in xo)
            ltx = jnp.maximum(bx1o, x1ob)
            lty = jnp.maximum(by1o, y1ob)
            rbx = jnp.minimum(bx2o, x2ob)
            rby = jnp.minimum(by2o, y2ob)
            iw = jnp.clip(rbx - ltx, 0.0, None)
            ih = jnp.clip(rby - lty, 0.0, None)
            inter = iw * ih
            iou = inter / (barea + areaob - inter + 1e-9)

            emit = s > _CONF_THRES
            kill = (iou > _IOU_THRES) | (idx == m)
            new.append(jnp.where(jnp.logical_and(emit, kill), neg, sc))

            row = jnp.where(
                lane == 0,
                ext[5, 0],
                jnp.where(
                    lane == 1,
                    ext[6, 0],
                    jnp.where(
                        lane == 2,
                        ext[7, 0],
                        jnp.where(
                            lane == 3,
                            ext[8, 0],
                            jnp.where(lane == 4, s, jnp.where(lane == 5, ext[9, 0], 0.0)),
                        ),
                    ),
                ),
            )
            o_ref[b, pl.ds(t, 1), :] = jnp.where(emit, row, 0.0)
        return tuple(new)

    lax.fori_loop(0, _MAX_DET, it, tuple(scores0[b] for b in range(nb)))


def kernel(x):
    pred = x[0]
    b = pred.shape[0]
    predt = jnp.transpose(pred, (0, 2, 1))
    predt = jnp.pad(predt, ((0, 0), (0, 0), (0, _NPAD - _N)))

    out = pl.pallas_call(
        _nms_body,
        out_shape=jax.ShapeDtypeStruct((b, _MAX_DET, _LANES), jnp.float32),
        scratch_shapes=[pltpu.VMEM((_NF, b * _ROWS, _LANES), jnp.float32)],
    )(predt)
    return out[:, :, :6]


# speedup vs baseline: 1.3430x; 1.3430x over previous
import jax
import jax.numpy as jnp
from jax import lax
from jax.experimental import pallas as pl
from jax.experimental.pallas import tpu as pltpu

_CONF_THRES = 0.001
_IOU_THRES = 0.45
_NC = 4
_MAX_WH = 4096.0
_MAX_DET = 300

_N = 5000
_NPAD = 5120
_ROWS = 40
_LANES = 128

_NF = 10


def _nms_body(x_ref, o_ref, f_ref):
    nb = x_ref.shape[0]

    def fld(c):
        return x_ref[:, c, :].reshape(-1, _ROWS, _LANES)

    cx, cy, w, h = fld(0), fld(1), fld(2), fld(3)
    obj = fld(4)
    x1 = cx - w / 2.0
    y1 = cy - h / 2.0
    x2 = cx + w / 2.0
    y2 = cy + h / 2.0

    c0 = fld(5) * obj
    c1 = fld(6) * obj
    c2 = fld(7) * obj
    c3 = fld(8) * obj
    conf = jnp.maximum(jnp.maximum(c0, c1), jnp.maximum(c2, c3))
    jf = jnp.where(
        c0 == conf,
        0.0,
        jnp.where(c1 == conf, 1.0, jnp.where(c2 == conf, 2.0, 3.0)),
    )

    iota0 = lax.broadcasted_iota(jnp.int32, (_ROWS, _LANES), 0)
    iota1 = lax.broadcasted_iota(jnp.int32, (_ROWS, _LANES), 1)
    idx0 = iota0 * _LANES + iota1
    scores0 = jnp.where((conf > _CONF_THRES) & (idx0 < _N)[None], conf, -1.0)

    off = jf * _MAX_WH
    x1o = x1 + off
    y1o = y1 + off
    x2o = x2 + off
    y2o = y2 + off
    areao = (x2o - x1o) * (y2o - y1o)

    for f, arr in enumerate([x1o, y1o, x2o, y2o, areao, x1, y1, x2, y2, jf]):
        f_ref[f] = arr.reshape(nb * _ROWS, _LANES)

    neg = jnp.float32(-jnp.inf)
    big = jnp.int32(1 << 30)

    def it(t, carry):
        lane = lax.broadcasted_iota(jnp.int32, (1, _LANES), 1)
        idx = lax.broadcasted_iota(jnp.int32, (_ROWS, _LANES), 0) * _LANES + lax.broadcasted_iota(
            jnp.int32, (_ROWS, _LANES), 1
        )
        new = []
        for b in range(nb):
            sc = carry[b]
            s = jnp.max(sc)
            m = jnp.min(jnp.where(sc == s, idx, big))
            sel = idx == m

            x1ob = f_ref[0, pl.ds(b * _ROWS, _ROWS), :]
            y1ob = f_ref[1, pl.ds(b * _ROWS, _ROWS), :]
            x2ob = f_ref[2, pl.ds(b * _ROWS, _ROWS), :]
            y2ob = f_ref[3, pl.ds(b * _ROWS, _ROWS), :]
            areaob = f_ref[4, pl.ds(b * _ROWS, _ROWS), :]

            def ext(arr):
                return jnp.sum(jnp.where(sel, arr, 0.0))

            bx1o = ext(x1ob)
            by1o = ext(y1ob)
            bx2o = ext(x2ob)
            by2o = ext(y2ob)
            barea = ext(areaob)

            ltx = jnp.maximum(bx1o, x1ob)
            lty = jnp.maximum(by1o, y1ob)
            rbx = jnp.minimum(bx2o, x2ob)
            rby = jnp.minimum(by2o, y2ob)
            iw = jnp.clip(rbx - ltx, 0.0, None)
            ih = jnp.clip(rby - lty, 0.0, None)
            inter = iw * ih
            iou = inter / (barea + areaob - inter + 1e-9)

            kill = (iou > _IOU_THRES) | sel
            new.append(jnp.where(kill, neg, sc))

            emit = s > _CONF_THRES
            row = jnp.where(
                lane == 0,
                ext(f_ref[5, pl.ds(b * _ROWS, _ROWS), :]),
                jnp.where(
                    lane == 1,
                    ext(f_ref[6, pl.ds(b * _ROWS, _ROWS), :]),
                    jnp.where(
                        lane == 2,
                        ext(f_ref[7, pl.ds(b * _ROWS, _ROWS), :]),
                        jnp.where(
                            lane == 3,
                            ext(f_ref[8, pl.ds(b * _ROWS, _ROWS), :]),
                            jnp.where(
                                lane == 4,
                                s,
                                jnp.where(
                                    lane == 5,
                                    ext(f_ref[9, pl.ds(b * _ROWS, _ROWS), :]),
                                    0.0,
                                ),
                            ),
                        ),
                    ),
                ),
            )
            o_ref[b, pl.ds(t, 1), :] = jnp.where(emit, row, 0.0)
        return tuple(new)

    lax.fori_loop(0, _MAX_DET, it, tuple(scores0[b] for b in range(nb)))


def kernel(x):
    pred = x[0]
    b = pred.shape[0]
    predt = jnp.transpose(pred, (0, 2, 1))
    predt = jnp.pad(predt, ((0, 0), (0, 0), (0, _NPAD - _N)))

    out = pl.pallas_call(
        _nms_body,
        out_shape=jax.ShapeDtypeStruct((b, _MAX_DET, _LANES), jnp.float32),
        scratch_shapes=[pltpu.VMEM((_NF, b * _ROWS, _LANES), jnp.float32)],
    )(predt)
    return out[:, :, :6]
